# baseline (speedup 1.0000x reference)
import functools

import jax
import jax.numpy as jnp
from jax import lax
from jax.experimental import pallas as pl
from jax.experimental.pallas import tpu as pltpu
from jax.experimental.pallas import tpu_sc as plsc

N = 50000
E = 800000
B = 32
T = 1000
DIM = 32
XD = 78
EMB = 128
OUT = 128
VOCAB = 26

NP = 50048
NROWS = 6248
NBIG = NROWS - 32 * 195
NTAIL = (E - NROWS * 128) // 128
CH = 128
SK = 5
NSUP = 195 // SK
ZR = 136
RPT = NP // 16

RBLK = 2000
NGBLK = N // RBLK

U = 8
G = T // U


def _edge_agg_body(h_hbm, src_hbm, dst_hbm, dst1_hbm, out_hbm,
                   shared, sidx, didx, rows, zbuf, gsem, ssem):
    c = lax.axis_index("c")
    s = lax.axis_index("s")
    wid = c * 16 + s

    def _z(i, _):
        zbuf[i, pl.ds(0, 16)] = jnp.zeros((16,), jnp.float32)
        zbuf[i, pl.ds(16, 16)] = jnp.zeros((16,), jnp.float32)
        return 0
    lax.fori_loop(0, ZR, _z, 0)
    for k in range(23):
        pltpu.sync_copy(zbuf, shared.at[pl.ds(s * RPT + k * ZR, ZR)])
    plsc.subcore_barrier()

    base_row = wid * 195 + jnp.minimum(wid, NBIG)

    def _sup(si, _):
        row0 = base_row + si * SK
        pltpu.sync_copy(src_hbm.at[pl.ds(row0 * CH, SK * CH)], sidx)
        gd = [pltpu.async_copy(h_hbm.at[sidx.at[pl.ds(j * CH, CH)]],
                               rows.at[j], gsem)
              for j in range(SK)]
        pltpu.sync_copy(dst_hbm.at[pl.ds(row0, SK)], didx)
        sd = []
        for j in range(SK):
            gd[j].wait()
            sd.append(pltpu.async_copy(rows.at[j], shared.at[didx.at[j]],
                                       ssem, add=True))
        for d in sd:
            d.wait()
        return 0
    lax.fori_loop(0, NSUP, _sup, 0)

    @pl.when(wid < NBIG)
    def _():
        row0 = base_row + 195
        pltpu.sync_copy(src_hbm.at[pl.ds(row0 * CH, CH)],
                        sidx.at[pl.ds(0, CH)])
        pltpu.async_copy(h_hbm.at[sidx.at[pl.ds(0, CH)]], rows.at[0],
                         gsem).wait()
        pltpu.sync_copy(dst_hbm.at[pl.ds(row0, 1)], didx.at[pl.ds(0, 1)])
        pltpu.async_copy(rows.at[0], shared.at[didx.at[0]], ssem,
                         add=True).wait()

    @pl.when((wid >= NBIG) & (wid < NBIG + NTAIL))
    def _():
        toff = NROWS * CH + (wid - NBIG) * CH
        pltpu.sync_copy(src_hbm.at[pl.ds(toff, CH)], sidx.at[pl.ds(0, CH)])
        pltpu.async_copy(h_hbm.at[sidx.at[pl.ds(0, CH)]], rows.at[0],
                         gsem).wait()
        pltpu.sync_copy(dst1_hbm.at[pl.ds(toff, CH)], didx.at[0])
        pltpu.async_copy(rows.at[0], shared.at[didx.at[0]], ssem,
                         add=True).wait()

    plsc.subcore_barrier()
    pltpu.sync_copy(shared.at[pl.ds(s * RPT, RPT)],
                    out_hbm.at[c, pl.ds(s * RPT, RPT)])


def _edge_agg(h, src1d, dst2d, dst1d):
    mesh = plsc.VectorSubcoreMesh(core_axis_name="c", subcore_axis_name="s")
    return pl.kernel(
        _edge_agg_body,
        out_type=jax.ShapeDtypeStruct((2, NP, DIM), jnp.float32),
        mesh=mesh,
        scratch_types=[
            pltpu.VMEM_SHARED((NP, DIM), jnp.float32),
            pltpu.VMEM((SK * CH,), jnp.int32),
            pltpu.VMEM((SK, CH), jnp.int32),
            pltpu.VMEM((SK, CH, DIM), jnp.float32),
            pltpu.VMEM((ZR, DIM), jnp.float32),
            pltpu.SemaphoreType.DMA,
            pltpu.SemaphoreType.DMA,
        ],
        compiler_params=pltpu.CompilerParams(use_tc_tiling_on_sc=False),
    )(h, src1d, dst2d, dst1d)


def _xw_body(x_ref, w_ref, o_ref):
    o_ref[...] = lax.dot_general(x_ref[...], w_ref[...], (((1,), (1,)), ((), ())),
                                 preferred_element_type=jnp.float32)


def _xw(x, w1):
    return pl.pallas_call(
        _xw_body,
        grid=(NGBLK,),
        in_specs=[pl.BlockSpec((RBLK, XD), lambda g: (g, 0)),
                  pl.BlockSpec((DIM, XD), lambda g: (0, 0))],
        out_specs=pl.BlockSpec((RBLK, DIM), lambda g: (g, 0)),
        out_shape=jax.ShapeDtypeStruct((N, DIM), jnp.float32),
    )(x, w1)


def _gin_body(t_ref, agg_ref, w1_ref, b1_ref, w2_ref, b2_ref, gs_ref, gb_ref,
              batch_ref, h_ref, pool_ref):
    t = t_ref[...] + agg_ref[0] + agg_ref[1]
    z = lax.dot_general(t, w1_ref[...], (((1,), (1,)), ((), ())),
                        preferred_element_type=jnp.float32) + b1_ref[...]
    z = jnp.maximum(z, 0.0)
    z = lax.dot_general(z, w2_ref[...], (((1,), (1,)), ((), ())),
                        preferred_element_type=jnp.float32) + b2_ref[...]
    h = jnp.maximum(z, 0.0) * gs_ref[...] + gb_ref[...]
    h_ref[...] = h
    ids = batch_ref[...][:, 0]
    iota_g = lax.broadcasted_iota(jnp.int32, (RBLK, B), 1)
    m = (ids[:, None] == iota_g).astype(jnp.float32)
    contrib = lax.dot_general(m, h, (((0,), (0,)), ((), ())),
                              preferred_element_type=jnp.float32)

    @pl.when(pl.program_id(0) == 0)
    def _():
        pool_ref[...] = jnp.zeros_like(pool_ref)
    pool_ref[...] += contrib


def _gin_tc(t_in, agg2, w1, b1, w2, b2, gs, gb, batch2d):
    return pl.pallas_call(
        _gin_body,
        grid=(NGBLK,),
        in_specs=[
            pl.BlockSpec((RBLK, DIM), lambda g: (g, 0)),
            pl.BlockSpec((2, RBLK, DIM), lambda g: (0, g, 0)),
            pl.BlockSpec((DIM, DIM), lambda g: (0, 0)),
            pl.BlockSpec((1, DIM), lambda g: (0, 0)),
            pl.BlockSpec((DIM, DIM), lambda g: (0, 0)),
            pl.BlockSpec((1, DIM), lambda g: (0, 0)),
            pl.BlockSpec((1, DIM), lambda g: (0, 0)),
            pl.BlockSpec((1, DIM), lambda g: (0, 0)),
            pl.BlockSpec((RBLK, 1), lambda g: (g, 0)),
        ],
        out_specs=[pl.BlockSpec((RBLK, DIM), lambda g: (g, 0)),
                   pl.BlockSpec((B, DIM), lambda g: (0, 0))],
        out_shape=[jax.ShapeDtypeStruct((N, DIM), jnp.float32),
                   jax.ShapeDtypeStruct((B, DIM), jnp.float32)],
    )(t_in, agg2, w1, b1, w2, b2, gs, gb, batch2d)


def _lstm_cell(gates, c_prev):
    ii = gates[:, 0 * EMB:1 * EMB]
    ff = gates[:, 1 * EMB:2 * EMB]
    gg = gates[:, 2 * EMB:3 * EMB]
    oo = gates[:, 3 * EMB:4 * EMB]
    c_new = jax.nn.sigmoid(ff) * c_prev + jax.nn.sigmoid(ii) * jnp.tanh(gg)
    h_new = jax.nn.sigmoid(oo) * jnp.tanh(c_new)
    return h_new, c_new


def _lstm0_body(tgtf_ref, tgtb_ref, emb_ref, wihf_ref, whhf_ref, bf_ref,
                wihb_ref, whhb_ref, bb_ref, h0_ref, c0_ref,
                out_ref, carry_ref, ew_ref):
    g = pl.program_id(0)

    @pl.when(g == 0)
    def _():
        carry_ref[0] = h0_ref[0]
        carry_ref[1] = c0_ref[0]
        carry_ref[2] = h0_ref[1]
        carry_ref[3] = c0_ref[1]
        ew_ref[0] = lax.dot_general(emb_ref[...], wihf_ref[...],
                                    (((1,), (1,)), ((), ())),
                                    preferred_element_type=jnp.float32)
        ew_ref[1] = lax.dot_general(emb_ref[...], wihb_ref[...],
                                    (((1,), (1,)), ((), ())),
                                    preferred_element_type=jnp.float32)

    hf, cf = carry_ref[0], carry_ref[1]
    hb, cb = carry_ref[2], carry_ref[3]
    ewf, ewb = ew_ref[0], ew_ref[1]
    iota_v = lax.broadcasted_iota(jnp.int32, (B, B), 1)
    for u in range(U):
        tf = tgtf_ref[u, :]
        onef = (tf[:, None] == iota_v).astype(jnp.float32)
        gates_f = (lax.dot_general(onef, ewf, (((1,), (0,)), ((), ())),
                                   preferred_element_type=jnp.float32)
                   + lax.dot_general(hf, whhf_ref[...], (((1,), (1,)), ((), ())),
                                     preferred_element_type=jnp.float32)
                   + bf_ref[...])
        hf, cf = _lstm_cell(gates_f, cf)
        out_ref[u, 0:B, :] = hf

        tb = tgtb_ref[U - 1 - u, :]
        oneb = (tb[:, None] == iota_v).astype(jnp.float32)
        gates_b = (lax.dot_general(oneb, ewb, (((1,), (0,)), ((), ())),
                                   preferred_element_type=jnp.float32)
                   + lax.dot_general(hb, whhb_ref[...], (((1,), (1,)), ((), ())),
                                     preferred_element_type=jnp.float32)
                   + bb_ref[...])
        hb, cb = _lstm_cell(gates_b, cb)
        out_ref[u, B:2 * B, :] = hb

    carry_ref[0], carry_ref[1] = hf, cf
    carry_ref[2], carry_ref[3] = hb, cb


def _lstm0(tgtT, emb_pad, wihf, whhf, bf, wihb, whhb, bb, h0, c0):
    return pl.pallas_call(
        _lstm0_body,
        grid=(G,),
        in_specs=[
            pl.BlockSpec((U, B), lambda g: (g, 0)),
            pl.BlockSpec((U, B), lambda g: (G - 1 - g, 0)),
            pl.BlockSpec((B, EMB), lambda g: (0, 0)),
            pl.BlockSpec((4 * EMB, EMB), lambda g: (0, 0)),
            pl.BlockSpec((4 * EMB, EMB), lambda g: (0, 0)),
            pl.BlockSpec((1, 4 * EMB), lambda g: (0, 0)),
            pl.BlockSpec((4 * EMB, EMB), lambda g: (0, 0)),
            pl.BlockSpec((4 * EMB, EMB), lambda g: (0, 0)),
            pl.BlockSpec((1, 4 * EMB), lambda g: (0, 0)),
            pl.BlockSpec((2, B, EMB), lambda g: (0, 0, 0)),
            pl.BlockSpec((2, B, EMB), lambda g: (0, 0, 0)),
        ],
        out_specs=pl.BlockSpec((U, 2 * B, EMB), lambda g: (g, 0, 0)),
        out_shape=jax.ShapeDtypeStruct((T, 2 * B, EMB), jnp.float32),
        scratch_shapes=[pltpu.VMEM((4, B, EMB), jnp.float32),
                        pltpu.VMEM((2, B, 4 * EMB), jnp.float32)],
    )(tgtT, tgtT, emb_pad, wihf, whhf, bf, wihb, whhb, bb, h0, c0)


def _lstm1_body(ysf_ref, ysr_ref, wihf_ref, whhf_ref, bf_ref,
                wihb_ref, whhb_ref, bb_ref, h0_ref, c0_ref,
                out_ref, carry_ref):
    g = pl.program_id(0)

    @pl.when(g == 0)
    def _():
        carry_ref[0] = h0_ref[0]
        carry_ref[1] = c0_ref[0]
        carry_ref[2] = h0_ref[1]
        carry_ref[3] = c0_ref[1]

    hf, cf = carry_ref[0], carry_ref[1]
    hb, cb = carry_ref[2], carry_ref[3]
    for u in range(U):
        inp_f = jnp.concatenate([ysf_ref[u, 0:B, :],
                                 ysr_ref[U - 1 - u, B:2 * B, :]], axis=1)
        gates_f = (lax.dot_general(inp_f, wihf_ref[...], (((1,), (1,)), ((), ())),
                                   preferred_element_type=jnp.float32)
                   + lax.dot_general(hf, whhf_ref[...], (((1,), (1,)), ((), ())),
                                     preferred_element_type=jnp.float32)
                   + bf_ref[...])
        hf, cf = _lstm_cell(gates_f, cf)
        out_ref[u, 0:B, :] = hf

        inp_b = jnp.concatenate([ysr_ref[U - 1 - u, 0:B, :],
                                 ysf_ref[u, B:2 * B, :]], axis=1)
        gates_b = (lax.dot_general(inp_b, wihb_ref[...], (((1,), (1,)), ((), ())),
                                   preferred_element_type=jnp.float32)
                   + lax.dot_general(hb, whhb_ref[...], (((1,), (1,)), ((), ())),
                                     preferred_element_type=jnp.float32)
                   + bb_ref[...])
        hb, cb = _lstm_cell(gates_b, cb)
        out_ref[u, B:2 * B, :] = hb

    carry_ref[0], carry_ref[1] = hf, cf
    carry_ref[2], carry_ref[3] = hb, cb


def _lstm1(ys0, wihf, whhf, bf, wihb, whhb, bb, h0, c0):
    return pl.pallas_call(
        _lstm1_body,
        grid=(G,),
        in_specs=[
            pl.BlockSpec((U, 2 * B, EMB), lambda g: (g, 0, 0)),
            pl.BlockSpec((U, 2 * B, EMB), lambda g: (G - 1 - g, 0, 0)),
            pl.BlockSpec((4 * EMB, 2 * EMB), lambda g: (0, 0)),
            pl.BlockSpec((4 * EMB, EMB), lambda g: (0, 0)),
            pl.BlockSpec((1, 4 * EMB), lambda g: (0, 0)),
            pl.BlockSpec((4 * EMB, 2 * EMB), lambda g: (0, 0)),
            pl.BlockSpec((4 * EMB, EMB), lambda g: (0, 0)),
            pl.BlockSpec((1, 4 * EMB), lambda g: (0, 0)),
            pl.BlockSpec((2, B, EMB), lambda g: (0, 0, 0)),
            pl.BlockSpec((2, B, EMB), lambda g: (0, 0, 0)),
        ],
        out_specs=pl.BlockSpec((U, 2 * B, EMB), lambda g: (g, 0, 0)),
        out_shape=jax.ShapeDtypeStruct((T, 2 * B, EMB), jnp.float32),
        scratch_shapes=[pltpu.VMEM((4, B, EMB), jnp.float32)],
    )(ys0, ys0, wihf, whhf, bf, wihb, whhb, bb, h0, c0)


def _fcxt_body(ysf_ref, ysr_ref, wr_ref, b_ref, o_ref, acc_ref):
    g = pl.program_id(0)

    @pl.when(g == 0)
    def _():
        acc_ref[...] = jnp.zeros_like(acc_ref)

    acc = acc_ref[...]
    for u in range(U):
        yfb = jnp.concatenate([ysf_ref[u, 0:B, :],
                               ysr_ref[U - 1 - u, B:2 * B, :]], axis=1)
        w_u = wr_ref[:, u, :]
        acc += lax.dot_general(yfb, w_u, (((1,), (1,)), ((), ())),
                               preferred_element_type=jnp.float32)
    acc_ref[...] = acc

    @pl.when(g == G - 1)
    def _():
        o_ref[...] = acc_ref[...] + b_ref[...]


def _fcxt(ys1, wr, b):
    return pl.pallas_call(
        _fcxt_body,
        grid=(G,),
        in_specs=[
            pl.BlockSpec((U, 2 * B, EMB), lambda g: (g, 0, 0)),
            pl.BlockSpec((U, 2 * B, EMB), lambda g: (G - 1 - g, 0, 0)),
            pl.BlockSpec((OUT, U, 2 * EMB), lambda g: (0, g, 0)),
            pl.BlockSpec((1, OUT), lambda g: (0, 0)),
        ],
        out_specs=pl.BlockSpec((B, OUT), lambda g: (0, 0)),
        out_shape=jax.ShapeDtypeStruct((B, OUT), jnp.float32),
        scratch_shapes=[pltpu.VMEM((B, OUT), jnp.float32)],
    )(ys1, ys1, wr, b)


def _head_body(gin_ref, gw_ref, gb_ref, xt_ref, w1_ref, b1_ref, w2_ref,
               b2_ref, ow_ref, ob_ref, o_ref):
    gg = lax.dot_general(gin_ref[...], gw_ref[...], (((1,), (1,)), ((), ())),
                         preferred_element_type=jnp.float32) + gb_ref[...]
    gg = jnp.maximum(gg, 0.0)
    xc = jnp.concatenate([gg, xt_ref[...]], axis=1)
    h1 = lax.dot_general(xc, w1_ref[...], (((1,), (1,)), ((), ())),
                         preferred_element_type=jnp.float32) + b1_ref[...]
    h1 = jnp.maximum(h1, 0.0)
    h2 = lax.dot_general(h1, w2_ref[...], (((1,), (1,)), ((), ())),
                         preferred_element_type=jnp.float32) + b2_ref[...]
    h2 = jnp.maximum(h2, 0.0)
    o_ref[...] = lax.dot_general(h2, ow_ref[...], (((1,), (1,)), ((), ())),
                                 preferred_element_type=jnp.float32) + ob_ref[...]


def _head(g_in, gw, gb, xt, w1, b1, w2, b2, ow, ob):
    return pl.pallas_call(
        _head_body,
        out_shape=jax.ShapeDtypeStruct((B, 128), jnp.float32),
    )(g_in, gw, gb, xt, w1, b1, w2, b2, ow, ob)


def kernel(x, edge_index, batch, target, hidden, cell, params):
    p = params
    src, dst = edge_index[0], edge_index[1]
    dst_p = dst[:NROWS * CH].reshape(NROWS, CH)
    batch2d = batch.reshape(N, 1)

    inv = 1.0 / jnp.sqrt(jnp.float32(1.0 + 1e-5))
    eye = jnp.eye(DIM, dtype=jnp.float32)

    h_cur = _xw(x, p['gin1_W1'])
    pools = []
    for i in range(1, 6):
        agg2 = _edge_agg(h_cur, src, dst_p, dst)
        w1 = eye if i == 1 else p['gin%d_W1' % i]
        h_cur, pool = _gin_tc(
            h_cur, agg2, w1,
            p['gin%d_b1' % i].reshape(1, DIM),
            p['gin%d_W2' % i], p['gin%d_b2' % i].reshape(1, DIM),
            (p['bn%d_g' % i] * inv).reshape(1, DIM),
            p['bn%d_b' % i].reshape(1, DIM), batch2d)
        pools.append(pool)
    g_in = jnp.concatenate(pools, axis=1)

    tgtT = target.T.astype(jnp.int32)
    emb_pad = jnp.zeros((B, EMB), jnp.float32).at[:VOCAB].set(p['emb'])
    b0f = (p['lstm0f_bih'] + p['lstm0f_bhh']).reshape(1, 4 * EMB)
    b0b = (p['lstm0b_bih'] + p['lstm0b_bhh']).reshape(1, 4 * EMB)
    b1f = (p['lstm1f_bih'] + p['lstm1f_bhh']).reshape(1, 4 * EMB)
    b1b = (p['lstm1b_bih'] + p['lstm1b_bhh']).reshape(1, 4 * EMB)
    ys0 = _lstm0(tgtT, emb_pad, p['lstm0f_Wih'], p['lstm0f_Whh'], b0f,
                 p['lstm0b_Wih'], p['lstm0b_Whh'], b0b,
                 hidden[0:2], cell[0:2])
    ys1 = _lstm1(ys0, p['lstm1f_Wih'], p['lstm1f_Whh'], b1f,
                 p['lstm1b_Wih'], p['lstm1b_Whh'], b1b,
                 hidden[2:4], cell[2:4])
    wr = p['fc_xt_W'].reshape(OUT, T, 2 * EMB)
    xt = _fcxt(ys1, wr, p['fc_xt_b'].reshape(1, OUT))

    ow_pad = jnp.zeros((128, 512), jnp.float32).at[0:1].set(p['out_W'])
    ob_pad = jnp.zeros((1, 128), jnp.float32).at[0, 0].set(p['out_b'][0])
    full = _head(g_in, p['fc_g1_W'], p['fc_g1_b'].reshape(1, OUT), xt,
                 p['fc1_W'], p['fc1_b'].reshape(1, 1024),
                 p['fc2_W'], p['fc2_b'].reshape(1, 512),
                 ow_pad, ob_pad)
    return full[:, 0:1]


# speedup vs baseline: 1.0065x; 1.0065x over previous
import functools

import jax
import jax.numpy as jnp
from jax import lax
from jax.experimental import pallas as pl
from jax.experimental.pallas import tpu as pltpu
from jax.experimental.pallas import tpu_sc as plsc

N = 50000
E = 800000
B = 32
T = 1000
DIM = 32
XD = 78
EMB = 128
OUT = 128
VOCAB = 26

NP = 50048
NROWS = E // 128
NBIG = NROWS - 32 * 195
CH = 128
SK = 5
NSUP = 195 // SK
ZR = 136
RPT = NP // 16

RBLK = 2000
NGBLK = N // RBLK

U = 8
G = T // U


def _edge_agg_body(h_hbm, src_hbm, dst_hbm, out_hbm,
                   shared, sidx, didx, rows, zbuf, gsem, ssem):
    c = lax.axis_index("c")
    s = lax.axis_index("s")
    wid = c * 16 + s

    def _z(i, _):
        zbuf[i, pl.ds(0, 16)] = jnp.zeros((16,), jnp.float32)
        zbuf[i, pl.ds(16, 16)] = jnp.zeros((16,), jnp.float32)
        return 0
    lax.fori_loop(0, ZR, _z, 0)
    for k in range(23):
        pltpu.sync_copy(zbuf, shared.at[pl.ds(s * RPT + k * ZR, ZR)])
    plsc.subcore_barrier()

    base_row = wid * 195 + jnp.minimum(wid, NBIG)

    def _sup(si, _):
        row0 = base_row + si * SK
        pltpu.sync_copy(src_hbm.at[pl.ds(row0, SK)], sidx)
        gd = [pltpu.async_copy(h_hbm.at[sidx.at[j]], rows.at[j], gsem)
              for j in range(SK)]
        pltpu.sync_copy(dst_hbm.at[pl.ds(row0, SK)], didx)
        sd = []
        for j in range(SK):
            gd[j].wait()
            sd.append(pltpu.async_copy(rows.at[j], shared.at[didx.at[j]],
                                       ssem, add=True))
        for d in sd:
            d.wait()
        return 0
    lax.fori_loop(0, NSUP, _sup, 0)

    @pl.when(wid < NBIG)
    def _():
        row0 = base_row + 195
        pltpu.sync_copy(src_hbm.at[pl.ds(row0, 1)], sidx.at[pl.ds(0, 1)])
        pltpu.async_copy(h_hbm.at[sidx.at[0]], rows.at[0], gsem).wait()
        pltpu.sync_copy(dst_hbm.at[pl.ds(row0, 1)], didx.at[pl.ds(0, 1)])
        pltpu.async_copy(rows.at[0], shared.at[didx.at[0]], ssem,
                         add=True).wait()

    plsc.subcore_barrier()
    pltpu.sync_copy(shared.at[pl.ds(s * RPT, RPT)],
                    out_hbm.at[c, pl.ds(s * RPT, RPT)])


def _edge_agg(h, src2d, dst2d):
    mesh = plsc.VectorSubcoreMesh(core_axis_name="c", subcore_axis_name="s")
    return pl.kernel(
        _edge_agg_body,
        out_type=jax.ShapeDtypeStruct((2, NP, DIM), jnp.float32),
        mesh=mesh,
        scratch_types=[
            pltpu.VMEM_SHARED((NP, DIM), jnp.float32),
            pltpu.VMEM((SK, CH), jnp.int32),
            pltpu.VMEM((SK, CH), jnp.int32),
            pltpu.VMEM((SK, CH, DIM), jnp.float32),
            pltpu.VMEM((ZR, DIM), jnp.float32),
            pltpu.SemaphoreType.DMA,
            pltpu.SemaphoreType.DMA,
        ],
        compiler_params=pltpu.CompilerParams(use_tc_tiling_on_sc=False),
    )(h, src2d, dst2d)


def _xw_body(x_ref, w_ref, o_ref):
    o_ref[...] = lax.dot_general(x_ref[...], w_ref[...], (((1,), (1,)), ((), ())),
                                 preferred_element_type=jnp.float32)


def _xw(x, w1):
    return pl.pallas_call(
        _xw_body,
        grid=(NGBLK,),
        in_specs=[pl.BlockSpec((RBLK, XD), lambda g: (g, 0)),
                  pl.BlockSpec((DIM, XD), lambda g: (0, 0))],
        out_specs=pl.BlockSpec((RBLK, DIM), lambda g: (g, 0)),
        out_shape=jax.ShapeDtypeStruct((N, DIM), jnp.float32),
    )(x, w1)


def _gin_body(t_ref, agg_ref, w1_ref, b1_ref, w2_ref, b2_ref, gs_ref, gb_ref,
              batch_ref, h_ref, pool_ref):
    t = t_ref[...] + agg_ref[0] + agg_ref[1]
    z = lax.dot_general(t, w1_ref[...], (((1,), (1,)), ((), ())),
                        preferred_element_type=jnp.float32) + b1_ref[...]
    z = jnp.maximum(z, 0.0)
    z = lax.dot_general(z, w2_ref[...], (((1,), (1,)), ((), ())),
                        preferred_element_type=jnp.float32) + b2_ref[...]
    h = jnp.maximum(z, 0.0) * gs_ref[...] + gb_ref[...]
    h_ref[...] = h
    ids = batch_ref[...][:, 0]
    iota_g = lax.broadcasted_iota(jnp.int32, (RBLK, B), 1)
    m = (ids[:, None] == iota_g).astype(jnp.float32)
    contrib = lax.dot_general(m, h, (((0,), (0,)), ((), ())),
                              preferred_element_type=jnp.float32)

    @pl.when(pl.program_id(0) == 0)
    def _():
        pool_ref[...] = jnp.zeros_like(pool_ref)
    pool_ref[...] += contrib


def _gin_tc(t_in, agg2, w1, b1, w2, b2, gs, gb, batch2d):
    return pl.pallas_call(
        _gin_body,
        grid=(NGBLK,),
        in_specs=[
            pl.BlockSpec((RBLK, DIM), lambda g: (g, 0)),
            pl.BlockSpec((2, RBLK, DIM), lambda g: (0, g, 0)),
            pl.BlockSpec((DIM, DIM), lambda g: (0, 0)),
            pl.BlockSpec((1, DIM), lambda g: (0, 0)),
            pl.BlockSpec((DIM, DIM), lambda g: (0, 0)),
            pl.BlockSpec((1, DIM), lambda g: (0, 0)),
            pl.BlockSpec((1, DIM), lambda g: (0, 0)),
            pl.BlockSpec((1, DIM), lambda g: (0, 0)),
            pl.BlockSpec((RBLK, 1), lambda g: (g, 0)),
        ],
        out_specs=[pl.BlockSpec((RBLK, DIM), lambda g: (g, 0)),
                   pl.BlockSpec((B, DIM), lambda g: (0, 0))],
        out_shape=[jax.ShapeDtypeStruct((N, DIM), jnp.float32),
                   jax.ShapeDtypeStruct((B, DIM), jnp.float32)],
    )(t_in, agg2, w1, b1, w2, b2, gs, gb, batch2d)


def _lstm_cell(gates, c_prev):
    ii = gates[:, 0 * EMB:1 * EMB]
    ff = gates[:, 1 * EMB:2 * EMB]
    gg = gates[:, 2 * EMB:3 * EMB]
    oo = gates[:, 3 * EMB:4 * EMB]
    c_new = jax.nn.sigmoid(ff) * c_prev + jax.nn.sigmoid(ii) * jnp.tanh(gg)
    h_new = jax.nn.sigmoid(oo) * jnp.tanh(c_new)
    return h_new, c_new


def _lstm0_body(tgtf_ref, tgtb_ref, emb_ref, wihf_ref, whhf_ref, bf_ref,
                wihb_ref, whhb_ref, bb_ref, h0_ref, c0_ref,
                out_ref, carry_ref, ew_ref):
    g = pl.program_id(0)

    @pl.when(g == 0)
    def _():
        carry_ref[0] = h0_ref[0]
        carry_ref[1] = c0_ref[0]
        carry_ref[2] = h0_ref[1]
        carry_ref[3] = c0_ref[1]
        ew_ref[0] = lax.dot_general(emb_ref[...], wihf_ref[...],
                                    (((1,), (1,)), ((), ())),
                                    preferred_element_type=jnp.float32)
        ew_ref[1] = lax.dot_general(emb_ref[...], wihb_ref[...],
                                    (((1,), (1,)), ((), ())),
                                    preferred_element_type=jnp.float32)

    hf, cf = carry_ref[0], carry_ref[1]
    hb, cb = carry_ref[2], carry_ref[3]
    ewf, ewb = ew_ref[0], ew_ref[1]
    iota_v = lax.broadcasted_iota(jnp.int32, (B, B), 1)
    for u in range(U):
        tf = tgtf_ref[u, :]
        onef = (tf[:, None] == iota_v).astype(jnp.float32)
        gates_f = (lax.dot_general(onef, ewf, (((1,), (0,)), ((), ())),
                                   preferred_element_type=jnp.float32)
                   + lax.dot_general(hf, whhf_ref[...], (((1,), (1,)), ((), ())),
                                     preferred_element_type=jnp.float32)
                   + bf_ref[...])
        hf, cf = _lstm_cell(gates_f, cf)
        out_ref[u, 0:B, :] = hf

        tb = tgtb_ref[U - 1 - u, :]
        oneb = (tb[:, None] == iota_v).astype(jnp.float32)
        gates_b = (lax.dot_general(oneb, ewb, (((1,), (0,)), ((), ())),
                                   preferred_element_type=jnp.float32)
                   + lax.dot_general(hb, whhb_ref[...], (((1,), (1,)), ((), ())),
                                     preferred_element_type=jnp.float32)
                   + bb_ref[...])
        hb, cb = _lstm_cell(gates_b, cb)
        out_ref[u, B:2 * B, :] = hb

    carry_ref[0], carry_ref[1] = hf, cf
    carry_ref[2], carry_ref[3] = hb, cb


def _lstm0(tgtT, emb_pad, wihf, whhf, bf, wihb, whhb, bb, h0, c0):
    return pl.pallas_call(
        _lstm0_body,
        grid=(G,),
        in_specs=[
            pl.BlockSpec((U, B), lambda g: (g, 0)),
            pl.BlockSpec((U, B), lambda g: (G - 1 - g, 0)),
            pl.BlockSpec((B, EMB), lambda g: (0, 0)),
            pl.BlockSpec((4 * EMB, EMB), lambda g: (0, 0)),
            pl.BlockSpec((4 * EMB, EMB), lambda g: (0, 0)),
            pl.BlockSpec((1, 4 * EMB), lambda g: (0, 0)),
            pl.BlockSpec((4 * EMB, EMB), lambda g: (0, 0)),
            pl.BlockSpec((4 * EMB, EMB), lambda g: (0, 0)),
            pl.BlockSpec((1, 4 * EMB), lambda g: (0, 0)),
            pl.BlockSpec((2, B, EMB), lambda g: (0, 0, 0)),
            pl.BlockSpec((2, B, EMB), lambda g: (0, 0, 0)),
        ],
        out_specs=pl.BlockSpec((U, 2 * B, EMB), lambda g: (g, 0, 0)),
        out_shape=jax.ShapeDtypeStruct((T, 2 * B, EMB), jnp.float32),
        scratch_shapes=[pltpu.VMEM((4, B, EMB), jnp.float32),
                        pltpu.VMEM((2, B, 4 * EMB), jnp.float32)],
    )(tgtT, tgtT, emb_pad, wihf, whhf, bf, wihb, whhb, bb, h0, c0)


def _lstm1_body(ysf_ref, ysr_ref, wihf_ref, whhf_ref, bf_ref,
                wihb_ref, whhb_ref, bb_ref, h0_ref, c0_ref,
                out_ref, carry_ref):
    g = pl.program_id(0)

    @pl.when(g == 0)
    def _():
        carry_ref[0] = h0_ref[0]
        carry_ref[1] = c0_ref[0]
        carry_ref[2] = h0_ref[1]
        carry_ref[3] = c0_ref[1]

    hf, cf = carry_ref[0], carry_ref[1]
    hb, cb = carry_ref[2], carry_ref[3]
    for u in range(U):
        inp_f = jnp.concatenate([ysf_ref[u, 0:B, :],
                                 ysr_ref[U - 1 - u, B:2 * B, :]], axis=1)
        gates_f = (lax.dot_general(inp_f, wihf_ref[...], (((1,), (1,)), ((), ())),
                                   preferred_element_type=jnp.float32)
                   + lax.dot_general(hf, whhf_ref[...], (((1,), (1,)), ((), ())),
                                     preferred_element_type=jnp.float32)
                   + bf_ref[...])
        hf, cf = _lstm_cell(gates_f, cf)
        out_ref[u, 0:B, :] = hf

        inp_b = jnp.concatenate([ysr_ref[U - 1 - u, 0:B, :],
                                 ysf_ref[u, B:2 * B, :]], axis=1)
        gates_b = (lax.dot_general(inp_b, wihb_ref[...], (((1,), (1,)), ((), ())),
                                   preferred_element_type=jnp.float32)
                   + lax.dot_general(hb, whhb_ref[...], (((1,), (1,)), ((), ())),
                                     preferred_element_type=jnp.float32)
                   + bb_ref[...])
        hb, cb = _lstm_cell(gates_b, cb)
        out_ref[u, B:2 * B, :] = hb

    carry_ref[0], carry_ref[1] = hf, cf
    carry_ref[2], carry_ref[3] = hb, cb


def _lstm1(ys0, wihf, whhf, bf, wihb, whhb, bb, h0, c0):
    return pl.pallas_call(
        _lstm1_body,
        grid=(G,),
        in_specs=[
            pl.BlockSpec((U, 2 * B, EMB), lambda g: (g, 0, 0)),
            pl.BlockSpec((U, 2 * B, EMB), lambda g: (G - 1 - g, 0, 0)),
            pl.BlockSpec((4 * EMB, 2 * EMB), lambda g: (0, 0)),
            pl.BlockSpec((4 * EMB, EMB), lambda g: (0, 0)),
            pl.BlockSpec((1, 4 * EMB), lambda g: (0, 0)),
            pl.BlockSpec((4 * EMB, 2 * EMB), lambda g: (0, 0)),
            pl.BlockSpec((4 * EMB, EMB), lambda g: (0, 0)),
            pl.BlockSpec((1, 4 * EMB), lambda g: (0, 0)),
            pl.BlockSpec((2, B, EMB), lambda g: (0, 0, 0)),
            pl.BlockSpec((2, B, EMB), lambda g: (0, 0, 0)),
        ],
        out_specs=pl.BlockSpec((U, 2 * B, EMB), lambda g: (g, 0, 0)),
        out_shape=jax.ShapeDtypeStruct((T, 2 * B, EMB), jnp.float32),
        scratch_shapes=[pltpu.VMEM((4, B, EMB), jnp.float32)],
    )(ys0, ys0, wihf, whhf, bf, wihb, whhb, bb, h0, c0)


def _fcxt_body(ysf_ref, ysr_ref, wr_ref, b_ref, o_ref, acc_ref):
    g = pl.program_id(0)

    @pl.when(g == 0)
    def _():
        acc_ref[...] = jnp.zeros_like(acc_ref)

    acc = acc_ref[...]
    for u in range(U):
        yfb = jnp.concatenate([ysf_ref[u, 0:B, :],
                               ysr_ref[U - 1 - u, B:2 * B, :]], axis=1)
        w_u = wr_ref[:, u, :]
        acc += lax.dot_general(yfb, w_u, (((1,), (1,)), ((), ())),
                               preferred_element_type=jnp.float32)
    acc_ref[...] = acc

    @pl.when(g == G - 1)
    def _():
        o_ref[...] = acc_ref[...] + b_ref[...]


def _fcxt(ys1, wr, b):
    return pl.pallas_call(
        _fcxt_body,
        grid=(G,),
        in_specs=[
            pl.BlockSpec((U, 2 * B, EMB), lambda g: (g, 0, 0)),
            pl.BlockSpec((U, 2 * B, EMB), lambda g: (G - 1 - g, 0, 0)),
            pl.BlockSpec((OUT, U, 2 * EMB), lambda g: (0, g, 0)),
            pl.BlockSpec((1, OUT), lambda g: (0, 0)),
        ],
        out_specs=pl.BlockSpec((B, OUT), lambda g: (0, 0)),
        out_shape=jax.ShapeDtypeStruct((B, OUT), jnp.float32),
        scratch_shapes=[pltpu.VMEM((B, OUT), jnp.float32)],
    )(ys1, ys1, wr, b)


def _head_body(gin_ref, gw_ref, gb_ref, xt_ref, w1_ref, b1_ref, w2_ref,
               b2_ref, ow_ref, ob_ref, o_ref):
    gg = lax.dot_general(gin_ref[...], gw_ref[...], (((1,), (1,)), ((), ())),
                         preferred_element_type=jnp.float32) + gb_ref[...]
    gg = jnp.maximum(gg, 0.0)
    xc = jnp.concatenate([gg, xt_ref[...]], axis=1)
    h1 = lax.dot_general(xc, w1_ref[...], (((1,), (1,)), ((), ())),
                         preferred_element_type=jnp.float32) + b1_ref[...]
    h1 = jnp.maximum(h1, 0.0)
    h2 = lax.dot_general(h1, w2_ref[...], (((1,), (1,)), ((), ())),
                         preferred_element_type=jnp.float32) + b2_ref[...]
    h2 = jnp.maximum(h2, 0.0)
    o_ref[...] = lax.dot_general(h2, ow_ref[...], (((1,), (1,)), ((), ())),
                                 preferred_element_type=jnp.float32) + ob_ref[...]


def _head(g_in, gw, gb, xt, w1, b1, w2, b2, ow, ob):
    return pl.pallas_call(
        _head_body,
        out_shape=jax.ShapeDtypeStruct((B, 128), jnp.float32),
    )(g_in, gw, gb, xt, w1, b1, w2, b2, ow, ob)


def kernel(x, edge_index, batch, target, hidden, cell, params):
    p = params
    src, dst = edge_index[0], edge_index[1]
    src_p = src.reshape(NROWS, CH)
    dst_p = dst.reshape(NROWS, CH)
    batch2d = batch.reshape(N, 1)

    inv = 1.0 / jnp.sqrt(jnp.float32(1.0 + 1e-5))
    eye = jnp.eye(DIM, dtype=jnp.float32)

    h_cur = _xw(x, p['gin1_W1'])
    pools = []
    for i in range(1, 6):
        agg2 = _edge_agg(h_cur, src_p, dst_p)
        w1 = eye if i == 1 else p['gin%d_W1' % i]
        h_cur, pool = _gin_tc(
            h_cur, agg2, w1,
            p['gin%d_b1' % i].reshape(1, DIM),
            p['gin%d_W2' % i], p['gin%d_b2' % i].reshape(1, DIM),
            (p['bn%d_g' % i] * inv).reshape(1, DIM),
            p['bn%d_b' % i].reshape(1, DIM), batch2d)
        pools.append(pool)
    g_in = jnp.concatenate(pools, axis=1)

    tgtT = target.T.astype(jnp.int32)
    emb_pad = jnp.zeros((B, EMB), jnp.float32).at[:VOCAB].set(p['emb'])
    b0f = (p['lstm0f_bih'] + p['lstm0f_bhh']).reshape(1, 4 * EMB)
    b0b = (p['lstm0b_bih'] + p['lstm0b_bhh']).reshape(1, 4 * EMB)
    b1f = (p['lstm1f_bih'] + p['lstm1f_bhh']).reshape(1, 4 * EMB)
    b1b = (p['lstm1b_bih'] + p['lstm1b_bhh']).reshape(1, 4 * EMB)
    ys0 = _lstm0(tgtT, emb_pad, p['lstm0f_Wih'], p['lstm0f_Whh'], b0f,
                 p['lstm0b_Wih'], p['lstm0b_Whh'], b0b,
                 hidden[0:2], cell[0:2])
    ys1 = _lstm1(ys0, p['lstm1f_Wih'], p['lstm1f_Whh'], b1f,
                 p['lstm1b_Wih'], p['lstm1b_Whh'], b1b,
                 hidden[2:4], cell[2:4])
    wr = p['fc_xt_W'].reshape(OUT, T, 2 * EMB)
    xt = _fcxt(ys1, wr, p['fc_xt_b'].reshape(1, OUT))

    ow_pad = jnp.zeros((128, 512), jnp.float32).at[0:1].set(p['out_W'])
    ob_pad = jnp.zeros((1, 128), jnp.float32).at[0, 0].set(p['out_b'][0])
    full = _head(g_in, p['fc_g1_W'], p['fc_g1_b'].reshape(1, OUT), xt,
                 p['fc1_W'], p['fc1_b'].reshape(1, 1024),
                 p['fc2_W'], p['fc2_b'].reshape(1, 512),
                 ow_pad, ob_pad)
    return full[:, 0:1]
